# baseline (device time: 443523 ns/iter reference)
import jax
import jax.numpy as jnp
from jax import lax
from jax.experimental import pallas as pl
from jax.experimental.pallas import tpu as pltpu

N_DEV = 32
E_TOTAL = 64
E_LOCAL = 2
CAP = 204
T = 512
D_IN = 256
D_FF = 512


def kernel(x, router_W, route_idx, expert_W):
    del router_W

    onehot = (route_idx == jnp.arange(E_TOTAL, dtype=jnp.int32)[None, :]).astype(
        jnp.int32
    )
    lrank = (
        jnp.sum(jnp.cumsum(onehot, axis=0) * onehot, axis=1, keepdims=True) - 1
    )
    cnt = jnp.sum(onehot, axis=0)
    cnt_pad = jnp.zeros((8, 128), jnp.int32).at[0, :E_TOTAL].set(cnt)

    def body(
        x_ref,
        ridx_ref,
        lrank_ref,
        cnt_ref,
        w_ref,
        out_ref,
        w_all,
        cnt_all,
        send_w,
        recv_w,
        send_c,
        recv_c,
    ):
        my = lax.axis_index("i")
        left = lax.rem(my + N_DEV - 1, N_DEV)
        right = lax.rem(my + 1, N_DEV)

        bar = pltpu.get_barrier_semaphore()
        pl.semaphore_signal(
            bar, inc=1, device_id=(left,), device_id_type=pl.DeviceIdType.MESH
        )
        pl.semaphore_signal(
            bar, inc=1, device_id=(right,), device_id_type=pl.DeviceIdType.MESH
        )
        pl.semaphore_wait(bar, 2)

        w_all[pl.ds(E_LOCAL * my, E_LOCAL), :, :] = w_ref[:, :, :]
        cnt_all[pl.ds(my, 1), :, :] = cnt_ref[:, :].reshape(1, 8, 128)

        for h in range(N_DEV - 1):
            o = lax.rem(my - h + 2 * N_DEV, N_DEV)
            ws = pl.ds(E_LOCAL * o, E_LOCAL)
            cs = pl.ds(o, 1)
            rw = pltpu.make_async_remote_copy(
                src_ref=w_all.at[ws],
                dst_ref=w_all.at[ws],
                send_sem=send_w.at[h],
                recv_sem=recv_w.at[h],
                device_id=(right,),
                device_id_type=pl.DeviceIdType.MESH,
            )
            rc = pltpu.make_async_remote_copy(
                src_ref=cnt_all.at[cs],
                dst_ref=cnt_all.at[cs],
                send_sem=send_c.at[h],
                recv_sem=recv_c.at[h],
                device_id=(right,),
                device_id_type=pl.DeviceIdType.MESH,
            )
            rw.start()
            rc.start()
            rw.wait()
            rc.wait()

        ridx = ridx_ref[:, :]
        cnts = cnt_all[:, 0, :E_TOTAL]
        sh = lax.broadcasted_iota(jnp.int32, (N_DEV, E_TOTAL), 0)
        prefix = jnp.sum(jnp.where(sh < my, cnts, 0), axis=0)
        eid = lax.broadcasted_iota(jnp.int32, (T, E_TOTAL), 1)
        oh = ridx == eid
        pref_tok = jnp.sum(
            jnp.where(oh, prefix[None, :], 0), axis=1, keepdims=True
        )
        keep = (lrank_ref[:, :] + pref_tok) < CAP

        xv = x_ref[:, :]
        out_ref[:, :] = jnp.zeros((T, D_FF), jnp.float32)

        def step(e_i, carry):
            sel = jnp.logical_and(ridx == e_i, keep)
            xm = jnp.where(sel, xv, 0.0)
            w = w_all[e_i]
            out_ref[:, :] += jnp.dot(xm, w, preferred_element_type=jnp.float32)
            return carry

        lax.fori_loop(0, E_TOTAL, step, 0)

    return pl.pallas_call(
        body,
        out_shape=jax.ShapeDtypeStruct((T, D_FF), jnp.float32),
        in_specs=[pl.BlockSpec(memory_space=pltpu.VMEM)] * 5,
        out_specs=pl.BlockSpec(memory_space=pltpu.VMEM),
        scratch_shapes=[
            pltpu.VMEM((E_TOTAL, D_IN, D_FF), jnp.float32),
            pltpu.VMEM((N_DEV, 8, 128), jnp.int32),
            pltpu.SemaphoreType.DMA((N_DEV,)),
            pltpu.SemaphoreType.DMA((N_DEV,)),
            pltpu.SemaphoreType.DMA((N_DEV,)),
            pltpu.SemaphoreType.DMA((N_DEV,)),
        ],
        compiler_params=pltpu.CompilerParams(
            collective_id=0, vmem_limit_bytes=100 * 1024 * 1024
        ),
    )(x, route_idx, lrank, cnt_pad, expert_W)


# device time: 117336 ns/iter; 3.7799x vs baseline; 3.7799x over previous
import jax
import jax.numpy as jnp
from jax import lax
from jax.experimental import pallas as pl
from jax.experimental.pallas import tpu as pltpu

N_DEV = 32
E_TOTAL = 64
E_LOCAL = 2
CAP = 204
SLOTS = 208
T = 512
D_IN = 256
D_FF = 512
NSLOT = E_LOCAL * SLOTS
XTRASH = NSLOT
XBUF_ROWS = NSLOT + 8
YTRASH = T
OBUF_ROWS = T + 8
BATCH = 16

_MESH = pl.DeviceIdType.MESH


def _exchange(cnt_pad, ridx_r):

    def body(cnt_ref, ridx_ref, cnt_all, ridx_all, csend, crecv, rsend, rrecv):
        my = lax.axis_index("i")
        bar = pltpu.get_barrier_semaphore()
        for j in range(1, N_DEV):
            p = lax.rem(my + j, N_DEV)
            pl.semaphore_signal(bar, inc=1, device_id=(p,), device_id_type=_MESH)
        pl.semaphore_wait(bar, N_DEV - 1)

        cnt_all[pl.ds(my, 1), :, :] = cnt_ref[:, :].reshape(1, 8, 128)
        ridx_all[pl.ds(my, 1), :, :] = ridx_ref[:, :].reshape(1, 4, 128)

        descs = []
        for j in range(1, N_DEV):
            p = lax.rem(my + j, N_DEV)
            dc = pltpu.make_async_remote_copy(
                src_ref=cnt_all.at[pl.ds(my, 1)],
                dst_ref=cnt_all.at[pl.ds(my, 1)],
                send_sem=csend.at[j],
                recv_sem=crecv.at[j],
                device_id=(p,),
                device_id_type=_MESH,
            )
            dr = pltpu.make_async_remote_copy(
                src_ref=ridx_all.at[pl.ds(my, 1)],
                dst_ref=ridx_all.at[pl.ds(my, 1)],
                send_sem=rsend.at[j],
                recv_sem=rrecv.at[j],
                device_id=(p,),
                device_id_type=_MESH,
            )
            dc.start()
            dr.start()
            descs.append((dc, dr))
        for dc, dr in descs:
            dc.wait_send()
            dr.wait_send()
            dc.wait_recv()
            dr.wait_recv()

    return pl.pallas_call(
        body,
        out_shape=[
            jax.ShapeDtypeStruct((N_DEV, 8, 128), jnp.int32),
            jax.ShapeDtypeStruct((N_DEV, 4, 128), jnp.int32),
        ],
        in_specs=[pl.BlockSpec(memory_space=pltpu.VMEM)] * 2,
        out_specs=[pl.BlockSpec(memory_space=pltpu.VMEM)] * 2,
        scratch_shapes=[
            pltpu.SemaphoreType.DMA((N_DEV,)),
            pltpu.SemaphoreType.DMA((N_DEV,)),
            pltpu.SemaphoreType.DMA((N_DEV,)),
            pltpu.SemaphoreType.DMA((N_DEV,)),
        ],
        compiler_params=pltpu.CompilerParams(collective_id=1),
    )(cnt_pad, ridx_r)


def _moe_a2a(x, expert_W, dest2, slot2, sm0, sm1, hdev2, hrow2, scal):

    def body(
        x_ref,
        w_ref,
        sm0_ref,
        sm1_ref,
        dest2_ref,
        slot2_ref,
        hdev2_ref,
        hrow2_ref,
        scal_ref,
        out_ref,
        xbuf,
        ybuf,
        obuf,
        dsend,
        drecv,
        csend,
        crecv,
    ):
        my = lax.axis_index("i")
        right = lax.rem(my + 1, N_DEV)
        obuf[:, :] = jnp.zeros((OBUF_ROWS, D_FF), jnp.float32)

        bar = pltpu.get_barrier_semaphore()
        for j in range(1, N_DEV):
            p = lax.rem(my + j, N_DEV)
            pl.semaphore_signal(bar, inc=1, device_id=(p,), device_id_type=_MESH)
        pl.semaphore_wait(bar, N_DEV - 1)

        n_disp_in = scal_ref[0]
        n_comb_in = scal_ref[1]
        f0 = scal_ref[2]
        f1 = scal_ref[3]

        def d_issue(par):
            def go(i, c):
                pltpu.make_async_remote_copy(
                    src_ref=x_ref.at[pl.ds(i, 1)],
                    dst_ref=xbuf.at[pl.ds(slot2_ref[i], 1)],
                    send_sem=dsend.at[par],
                    recv_sem=drecv,
                    device_id=(dest2_ref[i],),
                    device_id_type=_MESH,
                ).start()
                return c

            return go

        def d_dummy(par):
            return pltpu.make_async_remote_copy(
                src_ref=x_ref.at[pl.ds(0, 1)],
                dst_ref=xbuf.at[pl.ds(0, 1)],
                send_sem=dsend.at[par],
                recv_sem=drecv,
                device_id=(right,),
                device_id_type=_MESH,
            )

        n_batches = T // BATCH
        for b in range(n_batches):
            par = b % 2
            if b >= 2:
                lax.fori_loop(
                    0, BATCH, lambda i, c, par=par: (d_dummy(par).wait_send(), c)[1], 0
                )
            lax.fori_loop(b * BATCH, (b + 1) * BATCH, d_issue(par), 0)
        for par in (0, 1):
            lax.fori_loop(
                0, BATCH, lambda i, c, par=par: (d_dummy(par).wait_send(), c)[1], 0
            )

        lax.fori_loop(0, n_disp_in, lambda i, c: (d_dummy(0).wait_recv(), c)[1], 0)

        sio = lax.broadcasted_iota(jnp.int32, (NSLOT, 1), 0)
        lim = jnp.where(sio < SLOTS, f0, f1)
        valid = lax.rem(sio, SLOTS) < lim
        xb = jnp.where(valid, xbuf[0:NSLOT, :], 0.0)
        ybuf[0:SLOTS, :] = jnp.dot(
            xb[0:SLOTS, :], w_ref[0], preferred_element_type=jnp.float32
        )
        ybuf[SLOTS:, :] = jnp.dot(
            xb[SLOTS:, :], w_ref[1], preferred_element_type=jnp.float32
        )

        def c_issue(par):
            def go(s, c):
                pltpu.make_async_remote_copy(
                    src_ref=ybuf.at[pl.ds(s, 1)],
                    dst_ref=obuf.at[pl.ds(hrow2_ref[s], 1)],
                    send_sem=csend.at[par],
                    recv_sem=crecv,
                    device_id=(hdev2_ref[s],),
                    device_id_type=_MESH,
                ).start()
                return c

            return go

        def c_dummy(par):
            return pltpu.make_async_remote_copy(
                src_ref=ybuf.at[pl.ds(0, 1)],
                dst_ref=obuf.at[pl.ds(0, 1)],
                send_sem=csend.at[par],
                recv_sem=crecv,
                device_id=(right,),
                device_id_type=_MESH,
            )

        n_cbatches = NSLOT // BATCH
        for b in range(n_cbatches):
            par = b % 2
            if b >= 2:
                lax.fori_loop(
                    0, BATCH, lambda i, c, par=par: (c_dummy(par).wait_send(), c)[1], 0
                )
            lax.fori_loop(b * BATCH, (b + 1) * BATCH, c_issue(par), 0)
        for par in (0, 1):
            lax.fori_loop(
                0, BATCH, lambda i, c, par=par: (c_dummy(par).wait_send(), c)[1], 0
            )

        xv = x_ref[:, :]
        self_out = jnp.dot(
            xv * sm0_ref[:, :], w_ref[0], preferred_element_type=jnp.float32
        ) + jnp.dot(
            xv * sm1_ref[:, :], w_ref[1], preferred_element_type=jnp.float32
        )

        lax.fori_loop(0, n_comb_in, lambda i, c: (c_dummy(0).wait_recv(), c)[1], 0)

        out_ref[:, :] = obuf[0:T, :] + self_out

    smem = pl.BlockSpec(memory_space=pltpu.SMEM)
    vmem = pl.BlockSpec(memory_space=pltpu.VMEM)
    return pl.pallas_call(
        body,
        out_shape=jax.ShapeDtypeStruct((T, D_FF), jnp.float32),
        in_specs=[vmem, vmem, vmem, vmem, smem, smem, smem, smem, smem],
        out_specs=vmem,
        scratch_shapes=[
            pltpu.VMEM((XBUF_ROWS, D_IN), jnp.float32),
            pltpu.VMEM((NSLOT, D_FF), jnp.float32),
            pltpu.VMEM((OBUF_ROWS, D_FF), jnp.float32),
            pltpu.SemaphoreType.DMA((2,)),
            pltpu.SemaphoreType.DMA,
            pltpu.SemaphoreType.DMA((2,)),
            pltpu.SemaphoreType.DMA,
        ],
        compiler_params=pltpu.CompilerParams(collective_id=0),
    )(x, expert_W, sm0, sm1, dest2, slot2, hdev2, hrow2, scal)


def _metadata(route_idx, cnt_all, ridx_all):
    my = lax.axis_index("i")
    right = lax.rem(my + 1, N_DEV)
    e_tok = route_idx[:, 0]
    cnts = cnt_all[:, 0, :E_TOTAL]
    rflat = ridx_all.reshape(N_DEV, T)

    excl_shard = jnp.cumsum(cnts, axis=0) - cnts
    oh_all = (
        rflat[:, :, None] == jnp.arange(E_TOTAL, dtype=jnp.int32)[None, None, :]
    ).astype(jnp.int32)
    lrank_all = jnp.cumsum(oh_all, axis=1) - oh_all
    grank_all = jnp.sum(
        (excl_shard[:, None, :] + lrank_all) * oh_all, axis=2
    )
    keep_all = grank_all < CAP
    dest_all = rflat // E_LOCAL
    totals = jnp.sum(cnts, axis=0)
    filled = jnp.minimum(totals, CAP)
    filled2 = jnp.sum(filled.reshape(N_DEV, E_LOCAL), axis=1)
    shard_ids = jnp.arange(N_DEV, dtype=jnp.int32)[:, None]
    self_kept_per = jnp.sum(
        (keep_all & (dest_all == shard_ids)).astype(jnp.int32), axis=1
    )
    sent_away_per = jnp.sum(
        (keep_all & (dest_all != shard_ids)).astype(jnp.int32), axis=1
    )

    grank_my = lax.dynamic_slice_in_dim(grank_all, my, 1, 0)[0]
    keep_my = grank_my < CAP
    dest = e_tok // E_LOCAL
    slot = (e_tok % E_LOCAL) * SLOTS + jnp.minimum(grank_my, CAP - 1)
    is_local = dest == my
    real_send = keep_my & ~is_local
    dest2 = jnp.where(real_send, dest, right)
    slot2 = jnp.where(real_send, slot, XTRASH)
    selftok = keep_my & is_local
    n_out = jnp.sum(real_send.astype(jnp.int32))
    n_local = jnp.sum(selftok.astype(jnp.int32))

    gid_i = jnp.arange(N_DEV * T, dtype=jnp.int32)
    g2 = grank_all.reshape(-1)
    rf = rflat.reshape(-1)
    M = (
        g2[None, :] == jnp.arange(SLOTS, dtype=jnp.int32)[:, None]
    )
    homes = []
    for k in range(E_LOCAL):
        ek = E_LOCAL * my + k
        sel = (rf == ek) & (g2 < CAP)
        homes.append(
            jnp.sum(jnp.where(M & sel[None, :], gid_i[None, :], 0), axis=1)
        )
    home_gid = jnp.concatenate(homes).astype(jnp.int32)
    hdev = home_gid // T
    hrow = home_gid - hdev * T

    filled_my = lax.dynamic_slice_in_dim(filled, E_LOCAL * my, E_LOCAL, 0)
    slot_iota = jnp.arange(NSLOT, dtype=jnp.int32)
    slot_valid = (slot_iota % SLOTS) < jnp.where(
        slot_iota < SLOTS, filled_my[0], filled_my[1]
    )
    real_comb = slot_valid & (hdev != my)
    hdev2 = jnp.where(real_comb, hdev, right)
    hrow2 = jnp.where(real_comb, hrow, YTRASH)

    left = lax.rem(my + N_DEV - 1, N_DEV)
    kept_for_me = jnp.sum((keep_all & (dest_all == my)).astype(jnp.int32))
    n_disp_in = (kept_for_me - n_local) + (T - jnp.take(sent_away_per, left))
    n_comb_in = n_out + (
        NSLOT - (jnp.take(filled2, left) - jnp.take(self_kept_per, left))
    )
    scal = jnp.stack([n_disp_in, n_comb_in, filled_my[0], filled_my[1]]).astype(
        jnp.int32
    )

    sm0 = (selftok & (e_tok % E_LOCAL == 0)).astype(jnp.float32)[:, None]
    sm1 = (selftok & (e_tok % E_LOCAL == 1)).astype(jnp.float32)[:, None]
    return (
        dest2.astype(jnp.int32),
        slot2.astype(jnp.int32),
        sm0,
        sm1,
        hdev2.astype(jnp.int32),
        hrow2.astype(jnp.int32),
        scal,
    )


def kernel(x, router_W, route_idx, expert_W):
    del router_W

    onehot = (route_idx == jnp.arange(E_TOTAL, dtype=jnp.int32)[None, :]).astype(
        jnp.int32
    )
    cnt = jnp.sum(onehot, axis=0)
    cnt_pad = jnp.zeros((8, 128), jnp.int32).at[0, :E_TOTAL].set(cnt)
    ridx_r = route_idx[:, 0].reshape(4, 128)

    cnt_all, ridx_all = _exchange(cnt_pad, ridx_r)
    meta = _metadata(route_idx, cnt_all, ridx_all)
    return _moe_a2a(x, expert_W, *meta)


# device time: 78893 ns/iter; 5.6218x vs baseline; 1.4873x over previous
import jax
import jax.numpy as jnp
from jax import lax
from jax.experimental import pallas as pl
from jax.experimental.pallas import tpu as pltpu

N_DEV = 32
E_TOTAL = 64
E_LOCAL = 2
CAP = 204
SLOTS = 208
T = 512
D_IN = 256
D_FF = 512
NSLOT = E_LOCAL * SLOTS
XTRASH = NSLOT
XBUF_ROWS = NSLOT + 8
YTRASH = T
OBUF_ROWS = T + 8
BATCH = 16

_MESH = pl.DeviceIdType.MESH


def _exchange(cnt_pad, ridx_r):

    def body(cnt_ref, ridx_ref, cnt_all, ridx_all, csend, crecv, rsend, rrecv):
        my = lax.axis_index("i")
        bar = pltpu.get_barrier_semaphore()
        for j in range(1, N_DEV):
            p = lax.rem(my + j, N_DEV)
            pl.semaphore_signal(bar, inc=1, device_id=(p,), device_id_type=_MESH)
        pl.semaphore_wait(bar, N_DEV - 1)

        cnt_all[pl.ds(my, 1), :, :] = cnt_ref[:, :].reshape(1, 8, 128)
        ridx_all[pl.ds(my, 1), :, :] = ridx_ref[:, :].reshape(1, 4, 128)

        descs = []
        for j in range(1, N_DEV):
            p = lax.rem(my + j, N_DEV)
            dc = pltpu.make_async_remote_copy(
                src_ref=cnt_all.at[pl.ds(my, 1)],
                dst_ref=cnt_all.at[pl.ds(my, 1)],
                send_sem=csend.at[j],
                recv_sem=crecv.at[j],
                device_id=(p,),
                device_id_type=_MESH,
            )
            dr = pltpu.make_async_remote_copy(
                src_ref=ridx_all.at[pl.ds(my, 1)],
                dst_ref=ridx_all.at[pl.ds(my, 1)],
                send_sem=rsend.at[j],
                recv_sem=rrecv.at[j],
                device_id=(p,),
                device_id_type=_MESH,
            )
            dc.start()
            dr.start()
            descs.append((dc, dr))
        for dc, dr in descs:
            dc.wait_send()
            dr.wait_send()
            dc.wait_recv()
            dr.wait_recv()

    return pl.pallas_call(
        body,
        out_shape=[
            jax.ShapeDtypeStruct((N_DEV, 8, 128), jnp.int32),
            jax.ShapeDtypeStruct((N_DEV, 4, 128), jnp.int32),
        ],
        in_specs=[pl.BlockSpec(memory_space=pltpu.VMEM)] * 2,
        out_specs=[pl.BlockSpec(memory_space=pltpu.VMEM)] * 2,
        scratch_shapes=[
            pltpu.SemaphoreType.DMA((N_DEV,)),
            pltpu.SemaphoreType.DMA((N_DEV,)),
            pltpu.SemaphoreType.DMA((N_DEV,)),
            pltpu.SemaphoreType.DMA((N_DEV,)),
        ],
        compiler_params=pltpu.CompilerParams(collective_id=1),
    )(cnt_pad, ridx_r)


def _moe_a2a(x, expert_W, dest2, slot2, sm0, sm1, hdev2, hrow2, scal):

    def body(
        x_ref,
        w_ref,
        sm0_ref,
        sm1_ref,
        dest2_ref,
        slot2_ref,
        hdev2_ref,
        hrow2_ref,
        scal_ref,
        out_ref,
        xbuf,
        ybuf,
        obuf,
        dsend,
        drecv,
        csend,
        crecv,
    ):
        my = lax.axis_index("i")
        right = lax.rem(my + 1, N_DEV)
        obuf[:, :] = jnp.zeros((OBUF_ROWS, D_FF), jnp.float32)

        bar = pltpu.get_barrier_semaphore()
        for j in range(1, N_DEV):
            p = lax.rem(my + j, N_DEV)
            pl.semaphore_signal(bar, inc=1, device_id=(p,), device_id_type=_MESH)
        pl.semaphore_wait(bar, N_DEV - 1)

        n_disp_in = scal_ref[0]
        n_comb_in = scal_ref[1]
        f0 = scal_ref[2]
        f1 = scal_ref[3]

        def d_issue(par):
            def go(i, c):
                pltpu.make_async_remote_copy(
                    src_ref=x_ref.at[pl.ds(i, 1)],
                    dst_ref=xbuf.at[pl.ds(slot2_ref[i], 1)],
                    send_sem=dsend.at[par],
                    recv_sem=drecv,
                    device_id=(dest2_ref[i],),
                    device_id_type=_MESH,
                ).start()
                return c

            return go

        def d_dummy(par):
            return pltpu.make_async_remote_copy(
                src_ref=x_ref.at[pl.ds(0, 1)],
                dst_ref=xbuf.at[pl.ds(0, 1)],
                send_sem=dsend.at[par],
                recv_sem=drecv,
                device_id=(right,),
                device_id_type=_MESH,
            )

        n_batches = T // BATCH
        for b in range(n_batches):
            par = b % 2
            if b >= 2:
                lax.fori_loop(
                    0, BATCH, lambda i, c, par=par: (d_dummy(par).wait_send(), c)[1], 0
                )
            lax.fori_loop(b * BATCH, (b + 1) * BATCH, d_issue(par), 0)
        for par in (0, 1):
            lax.fori_loop(
                0, BATCH, lambda i, c, par=par: (d_dummy(par).wait_send(), c)[1], 0
            )

        lax.fori_loop(0, n_disp_in, lambda i, c: (d_dummy(0).wait_recv(), c)[1], 0)

        sio = lax.broadcasted_iota(jnp.int32, (NSLOT, 1), 0)
        lim = jnp.where(sio < SLOTS, f0, f1)
        valid = lax.rem(sio, SLOTS) < lim
        xb = jnp.where(valid, xbuf[0:NSLOT, :], 0.0)
        ybuf[0:SLOTS, :] = jnp.dot(
            xb[0:SLOTS, :], w_ref[0], preferred_element_type=jnp.float32
        )
        ybuf[SLOTS:, :] = jnp.dot(
            xb[SLOTS:, :], w_ref[1], preferred_element_type=jnp.float32
        )

        def c_issue(par):
            def go(s, c):
                pltpu.make_async_remote_copy(
                    src_ref=ybuf.at[pl.ds(s, 1)],
                    dst_ref=obuf.at[pl.ds(hrow2_ref[s], 1)],
                    send_sem=csend.at[par],
                    recv_sem=crecv,
                    device_id=(hdev2_ref[s],),
                    device_id_type=_MESH,
                ).start()
                return c

            return go

        def c_dummy(par):
            return pltpu.make_async_remote_copy(
                src_ref=ybuf.at[pl.ds(0, 1)],
                dst_ref=obuf.at[pl.ds(0, 1)],
                send_sem=csend.at[par],
                recv_sem=crecv,
                device_id=(right,),
                device_id_type=_MESH,
            )

        n_cbatches = NSLOT // BATCH
        for b in range(n_cbatches):
            par = b % 2
            if b >= 2:
                lax.fori_loop(
                    0, BATCH, lambda i, c, par=par: (c_dummy(par).wait_send(), c)[1], 0
                )
            lax.fori_loop(b * BATCH, (b + 1) * BATCH, c_issue(par), 0)
        for par in (0, 1):
            lax.fori_loop(
                0, BATCH, lambda i, c, par=par: (c_dummy(par).wait_send(), c)[1], 0
            )

        xv = x_ref[:, :]
        self_out = jnp.dot(
            xv * sm0_ref[:, :], w_ref[0], preferred_element_type=jnp.float32
        ) + jnp.dot(
            xv * sm1_ref[:, :], w_ref[1], preferred_element_type=jnp.float32
        )

        lax.fori_loop(0, n_comb_in, lambda i, c: (c_dummy(0).wait_recv(), c)[1], 0)

        out_ref[:, :] = obuf[0:T, :] + self_out

    smem = pl.BlockSpec(memory_space=pltpu.SMEM)
    vmem = pl.BlockSpec(memory_space=pltpu.VMEM)
    return pl.pallas_call(
        body,
        out_shape=jax.ShapeDtypeStruct((T, D_FF), jnp.float32),
        in_specs=[vmem, vmem, vmem, vmem, smem, smem, smem, smem, smem],
        out_specs=vmem,
        scratch_shapes=[
            pltpu.VMEM((XBUF_ROWS, D_IN), jnp.float32),
            pltpu.VMEM((NSLOT, D_FF), jnp.float32),
            pltpu.VMEM((OBUF_ROWS, D_FF), jnp.float32),
            pltpu.SemaphoreType.DMA((2,)),
            pltpu.SemaphoreType.DMA,
            pltpu.SemaphoreType.DMA((2,)),
            pltpu.SemaphoreType.DMA,
        ],
        compiler_params=pltpu.CompilerParams(collective_id=0),
    )(x, expert_W, sm0, sm1, dest2, slot2, hdev2, hrow2, scal)


def _metadata(route_idx, cnt_all, ridx_all):
    my = lax.axis_index("i")
    right = lax.rem(my + 1, N_DEV)
    e_tok = route_idx[:, 0]
    cnts = cnt_all[:, 0, :E_TOTAL]
    rflat = ridx_all.reshape(N_DEV, T)

    excl_shard = jnp.cumsum(cnts, axis=0) - cnts
    kept_se = jnp.minimum(jnp.maximum(CAP - excl_shard, 0), cnts)
    total_kept_per = jnp.sum(kept_se, axis=1)
    ids = jnp.arange(N_DEV)
    kept_pairs = kept_se.reshape(N_DEV, N_DEV, E_LOCAL)
    self_kept_per = jnp.sum(kept_pairs[ids, ids], axis=-1)
    sent_away_per = total_kept_per - self_kept_per
    totals = jnp.sum(cnts, axis=0)
    filled = jnp.minimum(totals, CAP)
    filled2 = jnp.sum(filled.reshape(N_DEV, E_LOCAL), axis=1)

    oh_my = (
        route_idx == jnp.arange(E_TOTAL, dtype=jnp.int32)[None, :]
    ).astype(jnp.int32)
    lrank_my = jnp.cumsum(oh_my, axis=0) - oh_my
    excl_my = lax.dynamic_slice_in_dim(excl_shard, my, 1, 0)
    grank_my = jnp.sum((excl_my + lrank_my) * oh_my, axis=1)
    keep_my = grank_my < CAP
    dest = e_tok // E_LOCAL
    slot = (e_tok % E_LOCAL) * SLOTS + jnp.minimum(grank_my, CAP - 1)
    is_local = dest == my
    real_send = keep_my & ~is_local
    dest2 = jnp.where(real_send, dest, right)
    slot2 = jnp.where(real_send, slot, XTRASH)
    selftok = keep_my & is_local
    n_local = jnp.take(self_kept_per, my)
    n_out = jnp.take(total_kept_per, my) - n_local

    gid_i = jnp.arange(N_DEV * T, dtype=jnp.int32)
    rf = rflat.reshape(-1)
    riota = jnp.arange(SLOTS, dtype=jnp.int32)[:, None]
    homes = []
    for k in range(E_LOCAL):
        ek = E_LOCAL * my + k
        mk = (rf == ek).astype(jnp.int32)
        exc = jnp.cumsum(mk) - mk
        sel = (mk == 1) & (exc < CAP)
        ind = (exc[None, :] == riota) & sel[None, :]
        homes.append(jnp.sum(jnp.where(ind, gid_i[None, :], 0), axis=1))
    home_gid = jnp.concatenate(homes).astype(jnp.int32)
    hdev = home_gid // T
    hrow = home_gid - hdev * T

    filled_my = lax.dynamic_slice_in_dim(filled, E_LOCAL * my, E_LOCAL, 0)
    slot_iota = jnp.arange(NSLOT, dtype=jnp.int32)
    slot_valid = (slot_iota % SLOTS) < jnp.where(
        slot_iota < SLOTS, filled_my[0], filled_my[1]
    )
    real_comb = slot_valid & (hdev != my)
    hdev2 = jnp.where(real_comb, hdev, right)
    hrow2 = jnp.where(real_comb, hrow, YTRASH)

    left = lax.rem(my + N_DEV - 1, N_DEV)
    kept_for_me = jnp.sum(filled_my)
    n_disp_in = (kept_for_me - n_local) + (T - jnp.take(sent_away_per, left))
    n_comb_in = n_out + (
        NSLOT - (jnp.take(filled2, left) - jnp.take(self_kept_per, left))
    )
    scal = jnp.stack([n_disp_in, n_comb_in, filled_my[0], filled_my[1]]).astype(
        jnp.int32
    )

    sm0 = (selftok & (e_tok % E_LOCAL == 0)).astype(jnp.float32)[:, None]
    sm1 = (selftok & (e_tok % E_LOCAL == 1)).astype(jnp.float32)[:, None]
    return (
        dest2.astype(jnp.int32),
        slot2.astype(jnp.int32),
        sm0,
        sm1,
        hdev2.astype(jnp.int32),
        hrow2.astype(jnp.int32),
        scal,
    )


def kernel(x, router_W, route_idx, expert_W):
    del router_W

    onehot = (route_idx == jnp.arange(E_TOTAL, dtype=jnp.int32)[None, :]).astype(
        jnp.int32
    )
    cnt = jnp.sum(onehot, axis=0)
    cnt_pad = jnp.zeros((8, 128), jnp.int32).at[0, :E_TOTAL].set(cnt)
    ridx_r = route_idx[:, 0].reshape(4, 128)

    cnt_all, ridx_all = _exchange(cnt_pad, ridx_r)
    meta = _metadata(route_idx, cnt_all, ridx_all)
    return _moe_a2a(x, expert_W, *meta)


# device time: 76612 ns/iter; 5.7892x vs baseline; 1.0298x over previous
import jax
import jax.numpy as jnp
from jax import lax
from jax.experimental import pallas as pl
from jax.experimental.pallas import tpu as pltpu

N_DEV = 32
E_TOTAL = 64
E_LOCAL = 2
CAP = 204
SLOTS = 208
T = 512
D_IN = 256
D_FF = 512
NSLOT = E_LOCAL * SLOTS
XTRASH = NSLOT
XBUF_ROWS = NSLOT + 8
YTRASH = T
OBUF_ROWS = T + 8
BATCH = 16

_MESH = pl.DeviceIdType.MESH


def _exchange(cnt_pad, ridx_r):

    def body(cnt_ref, ridx_ref, cnt_all, ridx_all, csend, crecv, rsend, rrecv):
        my = lax.axis_index("i")
        bar = pltpu.get_barrier_semaphore()
        for j in range(1, N_DEV):
            p = lax.rem(my + j, N_DEV)
            pl.semaphore_signal(bar, inc=1, device_id=(p,), device_id_type=_MESH)
        pl.semaphore_wait(bar, N_DEV - 1)

        cnt_all[pl.ds(my, 1), :, :] = cnt_ref[:, :].reshape(1, 8, 128)
        ridx_all[pl.ds(my, 1), :, :] = ridx_ref[:, :].reshape(1, 4, 128)

        descs = []
        for j in range(1, N_DEV):
            p = lax.rem(my + j, N_DEV)
            dc = pltpu.make_async_remote_copy(
                src_ref=cnt_all.at[pl.ds(my, 1)],
                dst_ref=cnt_all.at[pl.ds(my, 1)],
                send_sem=csend.at[j],
                recv_sem=crecv.at[j],
                device_id=(p,),
                device_id_type=_MESH,
            )
            dr = pltpu.make_async_remote_copy(
                src_ref=ridx_all.at[pl.ds(my, 1)],
                dst_ref=ridx_all.at[pl.ds(my, 1)],
                send_sem=rsend.at[j],
                recv_sem=rrecv.at[j],
                device_id=(p,),
                device_id_type=_MESH,
            )
            dc.start()
            dr.start()
            descs.append((dc, dr))
        for dc, dr in descs:
            dc.wait_send()
            dr.wait_send()
            dc.wait_recv()
            dr.wait_recv()

    return pl.pallas_call(
        body,
        out_shape=[
            jax.ShapeDtypeStruct((N_DEV, 8, 128), jnp.int32),
            jax.ShapeDtypeStruct((N_DEV, 4, 128), jnp.int32),
        ],
        in_specs=[pl.BlockSpec(memory_space=pltpu.VMEM)] * 2,
        out_specs=[pl.BlockSpec(memory_space=pltpu.VMEM)] * 2,
        scratch_shapes=[
            pltpu.SemaphoreType.DMA((N_DEV,)),
            pltpu.SemaphoreType.DMA((N_DEV,)),
            pltpu.SemaphoreType.DMA((N_DEV,)),
            pltpu.SemaphoreType.DMA((N_DEV,)),
        ],
        compiler_params=pltpu.CompilerParams(collective_id=1),
    )(cnt_pad, ridx_r)


def _moe_a2a(x, expert_W, dest2, slot2, sm0, sm1, hdev2, hrow2, scal):

    def body(
        x_ref,
        w_ref,
        sm0_ref,
        sm1_ref,
        dest2_ref,
        slot2_ref,
        hdev2_ref,
        hrow2_ref,
        scal_ref,
        out_ref,
        xbuf,
        ybuf,
        obuf,
        dsend,
        drecv,
        csend,
        crecv,
    ):
        my = lax.axis_index("i")
        right = lax.rem(my + 1, N_DEV)
        obuf[:, :] = jnp.zeros((OBUF_ROWS, D_FF), jnp.float32)

        bar = pltpu.get_barrier_semaphore()
        for j in range(1, N_DEV):
            p = lax.rem(my + j, N_DEV)
            pl.semaphore_signal(bar, inc=1, device_id=(p,), device_id_type=_MESH)
        pl.semaphore_wait(bar, N_DEV - 1)

        n_disp_in = scal_ref[0]
        n_comb_in = scal_ref[1]
        f0 = scal_ref[2]
        f1 = scal_ref[3]

        def d_issue(par):
            def go(i, c):
                pltpu.make_async_remote_copy(
                    src_ref=x_ref.at[pl.ds(i, 1)],
                    dst_ref=xbuf.at[pl.ds(slot2_ref[i], 1)],
                    send_sem=dsend.at[par],
                    recv_sem=drecv,
                    device_id=(dest2_ref[i],),
                    device_id_type=_MESH,
                ).start()
                return c

            return go

        def d_dummy(par, rows=1):
            return pltpu.make_async_remote_copy(
                src_ref=x_ref.at[pl.ds(0, rows)],
                dst_ref=xbuf.at[pl.ds(0, rows)],
                send_sem=dsend.at[par],
                recv_sem=drecv,
                device_id=(right,),
                device_id_type=_MESH,
            )

        n_batches = T // BATCH
        for b in range(n_batches):
            par = b % 2
            if b >= 2:
                d_dummy(par, BATCH).wait_send()
            lax.fori_loop(b * BATCH, (b + 1) * BATCH, d_issue(par), 0)
        for par in (0, 1):
            d_dummy(par, BATCH).wait_send()

        lax.fori_loop(
            0, n_disp_in // BATCH, lambda i, c: (d_dummy(0, BATCH).wait_recv(), c)[1], 0
        )
        lax.fori_loop(
            0, lax.rem(n_disp_in, BATCH), lambda i, c: (d_dummy(0).wait_recv(), c)[1], 0
        )

        sio = lax.broadcasted_iota(jnp.int32, (NSLOT, 1), 0)
        lim = jnp.where(sio < SLOTS, f0, f1)
        valid = lax.rem(sio, SLOTS) < lim
        xb = jnp.where(valid, xbuf[0:NSLOT, :], 0.0)
        ybuf[0:SLOTS, :] = jnp.dot(
            xb[0:SLOTS, :], w_ref[0], preferred_element_type=jnp.float32
        )
        ybuf[SLOTS:, :] = jnp.dot(
            xb[SLOTS:, :], w_ref[1], preferred_element_type=jnp.float32
        )

        def c_issue(par):
            def go(s, c):
                pltpu.make_async_remote_copy(
                    src_ref=ybuf.at[pl.ds(s, 1)],
                    dst_ref=obuf.at[pl.ds(hrow2_ref[s], 1)],
                    send_sem=csend.at[par],
                    recv_sem=crecv,
                    device_id=(hdev2_ref[s],),
                    device_id_type=_MESH,
                ).start()
                return c

            return go

        def c_dummy(par, rows=1):
            return pltpu.make_async_remote_copy(
                src_ref=ybuf.at[pl.ds(0, rows)],
                dst_ref=obuf.at[pl.ds(0, rows)],
                send_sem=csend.at[par],
                recv_sem=crecv,
                device_id=(right,),
                device_id_type=_MESH,
            )

        n_cbatches = NSLOT // BATCH
        for b in range(n_cbatches):
            par = b % 2
            if b >= 2:
                c_dummy(par, BATCH).wait_send()
            lax.fori_loop(b * BATCH, (b + 1) * BATCH, c_issue(par), 0)
        for par in (0, 1):
            c_dummy(par, BATCH).wait_send()

        xv = x_ref[:, :]
        self_out = jnp.dot(
            xv * sm0_ref[:, :], w_ref[0], preferred_element_type=jnp.float32
        ) + jnp.dot(
            xv * sm1_ref[:, :], w_ref[1], preferred_element_type=jnp.float32
        )

        lax.fori_loop(
            0, n_comb_in // BATCH, lambda i, c: (c_dummy(0, BATCH).wait_recv(), c)[1], 0
        )
        lax.fori_loop(
            0, lax.rem(n_comb_in, BATCH), lambda i, c: (c_dummy(0).wait_recv(), c)[1], 0
        )

        out_ref[:, :] = obuf[0:T, :] + self_out

    smem = pl.BlockSpec(memory_space=pltpu.SMEM)
    vmem = pl.BlockSpec(memory_space=pltpu.VMEM)
    return pl.pallas_call(
        body,
        out_shape=jax.ShapeDtypeStruct((T, D_FF), jnp.float32),
        in_specs=[vmem, vmem, vmem, vmem, smem, smem, smem, smem, smem],
        out_specs=vmem,
        scratch_shapes=[
            pltpu.VMEM((XBUF_ROWS, D_IN), jnp.float32),
            pltpu.VMEM((NSLOT, D_FF), jnp.float32),
            pltpu.VMEM((OBUF_ROWS, D_FF), jnp.float32),
            pltpu.SemaphoreType.DMA((2,)),
            pltpu.SemaphoreType.DMA,
            pltpu.SemaphoreType.DMA((2,)),
            pltpu.SemaphoreType.DMA,
        ],
        compiler_params=pltpu.CompilerParams(collective_id=0),
    )(x, expert_W, sm0, sm1, dest2, slot2, hdev2, hrow2, scal)


def _metadata(route_idx, cnt_all, ridx_all):
    my = lax.axis_index("i")
    right = lax.rem(my + 1, N_DEV)
    e_tok = route_idx[:, 0]
    cnts = cnt_all[:, 0, :E_TOTAL]
    rflat = ridx_all.reshape(N_DEV, T)

    excl_shard = jnp.cumsum(cnts, axis=0) - cnts
    kept_se = jnp.minimum(jnp.maximum(CAP - excl_shard, 0), cnts)
    total_kept_per = jnp.sum(kept_se, axis=1)
    ids = jnp.arange(N_DEV)
    kept_pairs = kept_se.reshape(N_DEV, N_DEV, E_LOCAL)
    self_kept_per = jnp.sum(kept_pairs[ids, ids], axis=-1)
    sent_away_per = total_kept_per - self_kept_per
    totals = jnp.sum(cnts, axis=0)
    filled = jnp.minimum(totals, CAP)
    filled2 = jnp.sum(filled.reshape(N_DEV, E_LOCAL), axis=1)

    oh_my = (
        route_idx == jnp.arange(E_TOTAL, dtype=jnp.int32)[None, :]
    ).astype(jnp.int32)
    lrank_my = jnp.cumsum(oh_my, axis=0) - oh_my
    excl_my = lax.dynamic_slice_in_dim(excl_shard, my, 1, 0)
    grank_my = jnp.sum((excl_my + lrank_my) * oh_my, axis=1)
    keep_my = grank_my < CAP
    dest = e_tok // E_LOCAL
    slot = (e_tok % E_LOCAL) * SLOTS + jnp.minimum(grank_my, CAP - 1)
    is_local = dest == my
    real_send = keep_my & ~is_local
    dest2 = jnp.where(real_send, dest, right)
    slot2 = jnp.where(real_send, slot, XTRASH)
    selftok = keep_my & is_local
    n_local = jnp.take(self_kept_per, my)
    n_out = jnp.take(total_kept_per, my) - n_local

    gid_i = jnp.arange(N_DEV * T, dtype=jnp.int32)
    rf = rflat.reshape(-1)
    riota = jnp.arange(SLOTS, dtype=jnp.int32)[:, None]
    homes = []
    for k in range(E_LOCAL):
        ek = E_LOCAL * my + k
        mk = (rf == ek).astype(jnp.int32)
        exc = jnp.cumsum(mk) - mk
        sel = (mk == 1) & (exc < CAP)
        ind = (exc[None, :] == riota) & sel[None, :]
        homes.append(jnp.sum(jnp.where(ind, gid_i[None, :], 0), axis=1))
    home_gid = jnp.concatenate(homes).astype(jnp.int32)
    hdev = home_gid // T
    hrow = home_gid - hdev * T

    filled_my = lax.dynamic_slice_in_dim(filled, E_LOCAL * my, E_LOCAL, 0)
    slot_iota = jnp.arange(NSLOT, dtype=jnp.int32)
    slot_valid = (slot_iota % SLOTS) < jnp.where(
        slot_iota < SLOTS, filled_my[0], filled_my[1]
    )
    real_comb = slot_valid & (hdev != my)
    hdev2 = jnp.where(real_comb, hdev, right)
    hrow2 = jnp.where(real_comb, hrow, YTRASH)

    left = lax.rem(my + N_DEV - 1, N_DEV)
    kept_for_me = jnp.sum(filled_my)
    n_disp_in = (kept_for_me - n_local) + (T - jnp.take(sent_away_per, left))
    n_comb_in = n_out + (
        NSLOT - (jnp.take(filled2, left) - jnp.take(self_kept_per, left))
    )
    scal = jnp.stack([n_disp_in, n_comb_in, filled_my[0], filled_my[1]]).astype(
        jnp.int32
    )

    sm0 = (selftok & (e_tok % E_LOCAL == 0)).astype(jnp.float32)[:, None]
    sm1 = (selftok & (e_tok % E_LOCAL == 1)).astype(jnp.float32)[:, None]
    return (
        dest2.astype(jnp.int32),
        slot2.astype(jnp.int32),
        sm0,
        sm1,
        hdev2.astype(jnp.int32),
        hrow2.astype(jnp.int32),
        scal,
    )


def kernel(x, router_W, route_idx, expert_W):
    del router_W

    onehot = (route_idx == jnp.arange(E_TOTAL, dtype=jnp.int32)[None, :]).astype(
        jnp.int32
    )
    cnt = jnp.sum(onehot, axis=0)
    cnt_pad = jnp.zeros((8, 128), jnp.int32).at[0, :E_TOTAL].set(cnt)
    ridx_r = route_idx[:, 0].reshape(4, 128)

    cnt_all, ridx_all = _exchange(cnt_pad, ridx_r)
    meta = _metadata(route_idx, cnt_all, ridx_all)
    return _moe_a2a(x, expert_W, *meta)
